# baseline (speedup 1.0000x reference)
import functools

import jax
import jax.numpy as jnp
from jax import lax
from jax.experimental import pallas as pl
from jax.experimental.pallas import tpu as pltpu
from jax.experimental.pallas import tpu_sc as plsc

IN_F = 4096
OUT_F = 8192
L = 16
R = 2
NACC = 4
NXB = 2
U = 8
NGROUPS = IN_F // L


def _make_sc_kernel(rows):
    NC, NS = 2, 16
    NW = NC * NS
    rows_per_w = rows // NW
    nbatch = rows_per_w // R
    mesh = plsc.VectorSubcoreMesh(core_axis_name="c", subcore_axis_name="s")

    @functools.partial(
        pl.kernel,
        mesh=mesh,
        compiler_params=pltpu.CompilerParams(needs_layout_passes=False),
        out_type=jax.ShapeDtypeStruct((rows * OUT_F,), jnp.float32),
        scratch_types=[
            pltpu.VMEM((IN_F,), jnp.int32),
            pltpu.VMEM((IN_F,), jnp.float32),
            pltpu.VMEM((NXB, R * IN_F), jnp.float32),
        ] + [
            pltpu.VMEM((R * OUT_F,), jnp.float32) for _ in range(NACC)
        ] + [
            pltpu.SemaphoreType.DMA,
            pltpu.SemaphoreType.DMA,
            pltpu.SemaphoreType.DMA,
            pltpu.SemaphoreType.DMA,
            pltpu.SemaphoreType.DMA,
            pltpu.SemaphoreType.DMA,
        ],
    )
    def k(x_hbm, h_hbm, s_hbm, out_hbm, h_v, s_v, x_v,
          a0, a1, a2, a3, sx0, sx1, so0, so1, so2, so3):
        acc = (a0, a1, a2, a3)
        sx = (sx0, sx1)
        so = (so0, so1, so2, so3)
        wid = lax.axis_index("s") * NC + lax.axis_index("c")
        base = wid * rows_per_w
        pltpu.sync_copy(h_hbm, h_v)
        pltpu.sync_copy(s_hbm, s_v)

        zero16 = jnp.zeros((L,), jnp.float32)
        row_off = [jnp.full((L,), r * OUT_F, jnp.int32) for r in range(R)]

        @plsc.parallel_loop(0, R * OUT_F // L, 1, unroll=8)
        def zinit(i):
            for ab in range(NACC):
                acc[ab][pl.ds(i * L, L)] = zero16

        def start_x(b, xb):
            pltpu.async_copy(
                x_hbm.at[pl.ds((base + b * R) * IN_F, R * IN_F)],
                x_v.at[xb], sx[xb])

        def wait_x(b, xb):
            pltpu.make_async_copy(
                x_hbm.at[pl.ds((base + b * R) * IN_F, R * IN_F)],
                x_v.at[xb], sx[xb]).wait()

        def start_out(b, ab):
            pltpu.async_copy(
                acc[ab], out_hbm.at[pl.ds((base + b * R) * OUT_F, R * OUT_F)],
                so[ab])

        def wait_out(b, ab):
            pltpu.make_async_copy(
                acc[ab], out_hbm.at[pl.ds((base + b * R) * OUT_F, R * OUT_F)],
                so[ab]).wait()

        def fused(xb, ab, cb):
            @plsc.parallel_loop(0, NGROUPS, 1, unroll=U)
            def body(i):
                off = i * L
                hv = h_v[pl.ds(off, L)]
                sv = s_v[pl.ds(off, L)]
                for r in range(R):
                    idx = hv + row_off[r] if r else hv
                    xv = x_v[xb, pl.ds(r * IN_F + off, L)]
                    plsc.addupdate_scatter(acc[ab], [idx], xv * sv)
                    if cb is not None:
                        plsc.store_scatter(acc[cb], [idx], zero16)

        start_x(0, 0)
        for b in range(NACC):
            wait_x(b, b % NXB)
            start_x(b + 1, (b + 1) % NXB)
            if b == NACC - 1:
                wait_out(b - 3, (b + 1) % NACC)
                fused(b % NXB, b % NACC, (b + 1) % NACC)
            else:
                fused(b % NXB, b % NACC, None)
            start_out(b, b % NACC)

        def superstep(ss, c):
            for u in range(NACC):
                b = ss * NACC + u
                wait_x(b, u % NXB)
                start_x(b + 1, (u + 1) % NXB)
                wait_out(b - 3, (u + 1) % NACC)
                fused(u % NXB, u, (u + 1) % NACC)
                start_out(b, u)
            return c

        lax.fori_loop(1, nbatch // NACC - 1, superstep, 0)

        for u in range(NACC):
            b = nbatch - NACC + u
            wait_x(b, u % NXB)
            if u < NACC - 1:
                start_x(b + 1, (u + 1) % NXB)
            wait_out(b - 3, (u + 1) % NACC)
            fused(u % NXB, u, (u + 1) % NACC if u < NACC - 1 else None)
            start_out(b, u)
        for u in range(1, NACC):
            wait_out(nbatch - NACC + u, u)

    return k


def kernel(x, h, s):
    lead = x.shape[:-1]
    rows = 1
    for d in lead:
        rows *= d
    x2 = x.reshape(rows * IN_F)
    out = _make_sc_kernel(rows)(x2, h, s)
    return out.reshape(lead + (OUT_F,))


# speedup vs baseline: 1.3859x; 1.3859x over previous
import functools

import jax
import jax.numpy as jnp
from jax import lax
from jax.experimental import pallas as pl
from jax.experimental.pallas import tpu as pltpu
from jax.experimental.pallas import tpu_sc as plsc

IN_F = 4096
OUT_F = 8192
L = 16
R = 2
NACC = 4
NXB = 2
U = 4
NGROUPS = IN_F // L


def _make_sc_kernel(rows):
    NC, NS = 2, 16
    NW = NC * NS
    rows_per_w = rows // NW
    nbatch = rows_per_w // R
    mesh = plsc.VectorSubcoreMesh(core_axis_name="c", subcore_axis_name="s")

    @functools.partial(
        pl.kernel,
        mesh=mesh,
        compiler_params=pltpu.CompilerParams(needs_layout_passes=False),
        out_type=jax.ShapeDtypeStruct((rows, OUT_F), jnp.float32),
        scratch_types=[
            pltpu.VMEM((IN_F,), jnp.int32),
            pltpu.VMEM((IN_F,), jnp.float32),
            pltpu.VMEM((NXB, R, IN_F), jnp.float32),
        ] + [
            pltpu.VMEM((OUT_F,), jnp.float32) for _ in range(NACC * R)
        ] + [
            pltpu.SemaphoreType.DMA,
            pltpu.SemaphoreType.DMA,
            pltpu.SemaphoreType.DMA,
            pltpu.SemaphoreType.DMA,
            pltpu.SemaphoreType.DMA,
            pltpu.SemaphoreType.DMA,
        ],
    )
    def k(x_hbm, h_hbm, s_hbm, out_hbm, h_v, s_v, x_v,
          a00, a01, a10, a11, a20, a21, a30, a31,
          sx0, sx1, so0, so1, so2, so3):
        acc = ((a00, a01), (a10, a11), (a20, a21), (a30, a31))
        sx = (sx0, sx1)
        so = (so0, so1, so2, so3)
        wid = lax.axis_index("s") * NC + lax.axis_index("c")
        base = wid * rows_per_w
        pltpu.sync_copy(h_hbm, h_v)
        pltpu.sync_copy(s_hbm, s_v)

        zero16 = jnp.zeros((L,), jnp.float32)

        @plsc.parallel_loop(0, OUT_F // L, 1, unroll=8)
        def zinit(i):
            for ab in range(NACC):
                for r in range(R):
                    acc[ab][r][pl.ds(i * L, L)] = zero16

        def start_x(b, xb):
            pltpu.async_copy(
                x_hbm.at[pl.ds(base + b * R, R)], x_v.at[xb], sx[xb])

        def wait_x(b, xb):
            pltpu.make_async_copy(
                x_hbm.at[pl.ds(base + b * R, R)], x_v.at[xb], sx[xb]).wait()

        def start_out(b, ab):
            for r in range(R):
                pltpu.async_copy(
                    acc[ab][r], out_hbm.at[base + b * R + r], so[ab])

        def wait_out(b, ab):
            for r in range(R):
                pltpu.make_async_copy(
                    acc[ab][r], out_hbm.at[base + b * R + r], so[ab]).wait()

        def fused(xb, ab, cb):
            @plsc.parallel_loop(0, NGROUPS, 1, unroll=U)
            def body(i):
                off = i * L
                hv = h_v[pl.ds(off, L)]
                sv = s_v[pl.ds(off, L)]
                for r in range(R):
                    xv = x_v[xb, r, pl.ds(off, L)]
                    plsc.addupdate_scatter(acc[ab][r], [hv], xv * sv)
                if cb is not None:
                    for r in range(R):
                        plsc.store_scatter(acc[cb][r], [hv], zero16)

        start_x(0, 0)
        for b in range(NACC):
            wait_x(b, b % NXB)
            start_x(b + 1, (b + 1) % NXB)
            if b == NACC - 1:
                wait_out(b - 3, (b + 1) % NACC)
                fused(b % NXB, b % NACC, (b + 1) % NACC)
            else:
                fused(b % NXB, b % NACC, None)
            start_out(b, b % NACC)

        def superstep(ss, c):
            for u in range(NACC):
                b = ss * NACC + u
                wait_x(b, u % NXB)
                start_x(b + 1, (u + 1) % NXB)
                wait_out(b - 3, (u + 1) % NACC)
                fused(u % NXB, u, (u + 1) % NACC)
                start_out(b, u)
            return c

        lax.fori_loop(1, nbatch // NACC - 1, superstep, 0)

        for u in range(NACC):
            b = nbatch - NACC + u
            wait_x(b, u % NXB)
            if u < NACC - 1:
                start_x(b + 1, (u + 1) % NXB)
            wait_out(b - 3, (u + 1) % NACC)
            fused(u % NXB, u, (u + 1) % NACC if u < NACC - 1 else None)
            start_out(b, u)
        for u in range(1, NACC):
            wait_out(nbatch - NACC + u, u)

    return k


ROWS_TC = 1024
BM = 2048


def _tc_body(h_ref, s_ref, x_ref, o_ref, w_ref):
    m = pl.program_id(0)
    cols = jax.lax.broadcasted_iota(jnp.int32, (IN_F, BM), 1) + m * BM
    w_ref[...] = jnp.where(
        h_ref[...][:, None] == cols, s_ref[...][:, None], 0.0
    ).astype(jnp.bfloat16)
    xb = x_ref[...].astype(jnp.bfloat16)
    o_ref[...] = jnp.dot(xb, w_ref[...], preferred_element_type=jnp.float32)


def _make_tc_kernel(rows_tc):
    return pl.pallas_call(
        _tc_body,
        grid=(OUT_F // BM,),
        in_specs=[
            pl.BlockSpec((IN_F,), lambda m: (0,)),
            pl.BlockSpec((IN_F,), lambda m: (0,)),
            pl.BlockSpec((rows_tc, IN_F), lambda m: (0, 0)),
        ],
        out_specs=pl.BlockSpec((rows_tc, BM), lambda m: (0, m)),
        out_shape=jax.ShapeDtypeStruct((rows_tc, OUT_F), jnp.float32),
        scratch_shapes=[pltpu.VMEM((IN_F, BM), jnp.bfloat16)],
    )


def kernel(x, h, s):
    lead = x.shape[:-1]
    rows = 1
    for d in lead:
        rows *= d
    x2 = x.reshape(rows, IN_F)
    rows_sc = rows - ROWS_TC
    out_sc = _make_sc_kernel(rows_sc)(x2[:rows_sc], h, s)
    out_tc = _make_tc_kernel(ROWS_TC)(h, s, x2[rows_sc:])
    out = jnp.concatenate([out_sc, out_tc], axis=0)
    return out.reshape(lead + (OUT_F,))


# speedup vs baseline: 2.6748x; 1.9300x over previous
import functools

import jax
import jax.numpy as jnp
from jax import lax
from jax.experimental import pallas as pl
from jax.experimental.pallas import tpu as pltpu
from jax.experimental.pallas import tpu_sc as plsc

IN_F = 4096
OUT_F = 8192
L = 16
R = 2
NACC = 4
NXB = 2
U = 4
NGROUPS = IN_F // L


def _make_sc_kernel(rows):
    NC, NS = 2, 16
    NW = NC * NS
    rows_per_w = rows // NW
    nbatch = rows_per_w // R
    mesh = plsc.VectorSubcoreMesh(core_axis_name="c", subcore_axis_name="s")

    @functools.partial(
        pl.kernel,
        mesh=mesh,
        compiler_params=pltpu.CompilerParams(needs_layout_passes=False),
        out_type=jax.ShapeDtypeStruct((rows, OUT_F), jnp.float32),
        scratch_types=[
            pltpu.VMEM((IN_F,), jnp.int32),
            pltpu.VMEM((IN_F,), jnp.float32),
            pltpu.VMEM((NXB, R, IN_F), jnp.float32),
        ] + [
            pltpu.VMEM((OUT_F,), jnp.float32) for _ in range(NACC * R)
        ] + [
            pltpu.SemaphoreType.DMA,
            pltpu.SemaphoreType.DMA,
            pltpu.SemaphoreType.DMA,
            pltpu.SemaphoreType.DMA,
            pltpu.SemaphoreType.DMA,
            pltpu.SemaphoreType.DMA,
        ],
    )
    def k(x_hbm, h_hbm, s_hbm, out_hbm, h_v, s_v, x_v,
          a00, a01, a10, a11, a20, a21, a30, a31,
          sx0, sx1, so0, so1, so2, so3):
        acc = ((a00, a01), (a10, a11), (a20, a21), (a30, a31))
        sx = (sx0, sx1)
        so = (so0, so1, so2, so3)
        wid = lax.axis_index("s") * NC + lax.axis_index("c")
        base = wid * rows_per_w
        pltpu.sync_copy(h_hbm, h_v)
        pltpu.sync_copy(s_hbm, s_v)

        zero16 = jnp.zeros((L,), jnp.float32)

        @plsc.parallel_loop(0, OUT_F // L, 1, unroll=8)
        def zinit(i):
            for ab in range(NACC):
                for r in range(R):
                    acc[ab][r][pl.ds(i * L, L)] = zero16

        def start_x(b, xb):
            pltpu.async_copy(
                x_hbm.at[pl.ds(base + b * R, R)], x_v.at[xb], sx[xb])

        def wait_x(b, xb):
            pltpu.make_async_copy(
                x_hbm.at[pl.ds(base + b * R, R)], x_v.at[xb], sx[xb]).wait()

        def start_out(b, ab):
            for r in range(R):
                pltpu.async_copy(
                    acc[ab][r], out_hbm.at[base + b * R + r], so[ab])

        def wait_out(b, ab):
            for r in range(R):
                pltpu.make_async_copy(
                    acc[ab][r], out_hbm.at[base + b * R + r], so[ab]).wait()

        def fused(xb, ab, cb):
            @plsc.parallel_loop(0, NGROUPS, 1, unroll=U)
            def body(i):
                off = i * L
                hv = h_v[pl.ds(off, L)]
                sv = s_v[pl.ds(off, L)]
                for r in range(R):
                    xv = x_v[xb, r, pl.ds(off, L)]
                    plsc.addupdate_scatter(acc[ab][r], [hv], xv * sv)
                if cb is not None:
                    for r in range(R):
                        plsc.store_scatter(acc[cb][r], [hv], zero16)

        start_x(0, 0)
        for b in range(NACC):
            wait_x(b, b % NXB)
            start_x(b + 1, (b + 1) % NXB)
            if b == NACC - 1:
                wait_out(b - 3, (b + 1) % NACC)
                fused(b % NXB, b % NACC, (b + 1) % NACC)
            else:
                fused(b % NXB, b % NACC, None)
            start_out(b, b % NACC)

        def superstep(ss, c):
            for u in range(NACC):
                b = ss * NACC + u
                wait_x(b, u % NXB)
                start_x(b + 1, (u + 1) % NXB)
                wait_out(b - 3, (u + 1) % NACC)
                fused(u % NXB, u, (u + 1) % NACC)
                start_out(b, u)
            return c

        lax.fori_loop(1, nbatch // NACC - 1, superstep, 0)

        for u in range(NACC):
            b = nbatch - NACC + u
            wait_x(b, u % NXB)
            if u < NACC - 1:
                start_x(b + 1, (u + 1) % NXB)
            wait_out(b - 3, (u + 1) % NACC)
            fused(u % NXB, u, (u + 1) % NACC if u < NACC - 1 else None)
            start_out(b, u)
        for u in range(1, NACC):
            wait_out(nbatch - NACC + u, u)

    return k


def kernel(x, h, s):
    lead = x.shape[:-1]
    rows = 1
    for d in lead:
        rows *= d
    x2 = x.reshape(rows, IN_F)
    out = _make_sc_kernel(rows)(x2, h, s)
    return out.reshape(lead + (OUT_F,))


# speedup vs baseline: 2.7507x; 1.0284x over previous
import functools

import jax
import jax.numpy as jnp
from jax import lax
from jax.experimental import pallas as pl
from jax.experimental.pallas import tpu as pltpu
from jax.experimental.pallas import tpu_sc as plsc

IN_F = 4096
OUT_F = 8192
L = 16
R = 2
NACC = 4
NXB = 2
U = 4
NGROUPS = IN_F // L


def _make_sc_kernel(rows):
    NC, NS = 2, 16
    NW = NC * NS
    rows_per_w = rows // NW
    nbatch = rows_per_w // R
    mesh = plsc.VectorSubcoreMesh(core_axis_name="c", subcore_axis_name="s")

    @functools.partial(
        pl.kernel,
        mesh=mesh,
        compiler_params=pltpu.CompilerParams(needs_layout_passes=False),
        out_type=jax.ShapeDtypeStruct((rows, OUT_F), jnp.float32),
        scratch_types=[
            pltpu.VMEM((IN_F,), jnp.int32),
            pltpu.VMEM((IN_F,), jnp.float32),
            pltpu.VMEM((NXB, R, IN_F), jnp.float32),
        ] + [
            pltpu.VMEM((OUT_F,), jnp.float32) for _ in range(NACC * R)
        ] + [
            pltpu.SemaphoreType.DMA,
            pltpu.SemaphoreType.DMA,
            pltpu.SemaphoreType.DMA,
            pltpu.SemaphoreType.DMA,
            pltpu.SemaphoreType.DMA,
            pltpu.SemaphoreType.DMA,
        ],
    )
    def k(x_hbm, h_hbm, s_hbm, out_hbm, h_v, s_v, x_v,
          a00, a01, a10, a11, a20, a21, a30, a31,
          sx0, sx1, so0, so1, so2, so3):
        acc = ((a00, a01), (a10, a11), (a20, a21), (a30, a31))
        sx = (sx0, sx1)
        so = (so0, so1, so2, so3)
        wid = lax.axis_index("s") * NC + lax.axis_index("c")
        base = wid * rows_per_w
        pltpu.async_copy(h_hbm, h_v, so[0])
        pltpu.async_copy(s_hbm, s_v, so[1])
        pltpu.async_copy(x_hbm.at[pl.ds(base, R)], x_v.at[0], sx[0])

        zero16 = jnp.zeros((L,), jnp.float32)

        @plsc.parallel_loop(0, OUT_F // L, 1, unroll=8)
        def zinit(i):
            for ab in range(NACC):
                for r in range(R):
                    acc[ab][r][pl.ds(i * L, L)] = zero16

        pltpu.make_async_copy(h_hbm, h_v, so[0]).wait()
        pltpu.make_async_copy(s_hbm, s_v, so[1]).wait()

        def start_x(b, xb):
            pltpu.async_copy(
                x_hbm.at[pl.ds(base + b * R, R)], x_v.at[xb], sx[xb])

        def wait_x(b, xb):
            pltpu.make_async_copy(
                x_hbm.at[pl.ds(base + b * R, R)], x_v.at[xb], sx[xb]).wait()

        def start_out(b, ab):
            for r in range(R):
                pltpu.async_copy(
                    acc[ab][r], out_hbm.at[base + b * R + r], so[ab])

        def wait_out(b, ab):
            for r in range(R):
                pltpu.make_async_copy(
                    acc[ab][r], out_hbm.at[base + b * R + r], so[ab]).wait()

        def fused(xb, ab, cb):
            @plsc.parallel_loop(0, NGROUPS, 1, unroll=U)
            def body(i):
                off = i * L
                hv = h_v[pl.ds(off, L)]
                sv = s_v[pl.ds(off, L)]
                for r in range(R):
                    xv = x_v[xb, r, pl.ds(off, L)]
                    plsc.addupdate_scatter(acc[ab][r], [hv], xv * sv)
                if cb is not None:
                    for r in range(R):
                        plsc.store_scatter(acc[cb][r], [hv], zero16)

        for b in range(NACC):
            wait_x(b, b % NXB)
            start_x(b + 1, (b + 1) % NXB)
            if b == NACC - 1:
                wait_out(b - 3, (b + 1) % NACC)
                fused(b % NXB, b % NACC, (b + 1) % NACC)
            else:
                fused(b % NXB, b % NACC, None)
            start_out(b, b % NACC)

        def superstep(ss, c):
            for u in range(NACC):
                b = ss * NACC + u
                wait_x(b, u % NXB)
                start_x(b + 1, (u + 1) % NXB)
                wait_out(b - 3, (u + 1) % NACC)
                fused(u % NXB, u, (u + 1) % NACC)
                start_out(b, u)
            return c

        lax.fori_loop(1, nbatch // NACC - 1, superstep, 0)

        for u in range(NACC):
            b = nbatch - NACC + u
            wait_x(b, u % NXB)
            if u < NACC - 1:
                start_x(b + 1, (u + 1) % NXB)
            wait_out(b - 3, (u + 1) % NACC)
            fused(u % NXB, u, (u + 1) % NACC if u < NACC - 1 else None)
            start_out(b, u)
        for u in range(1, NACC):
            wait_out(nbatch - NACC + u, u)

    return k


def kernel(x, h, s):
    lead = x.shape[:-1]
    rows = 1
    for d in lead:
        rows *= d
    x2 = x.reshape(rows, IN_F)
    out = _make_sc_kernel(rows)(x2, h, s)
    return out.reshape(lead + (OUT_F,))
